# baseline (speedup 1.0000x reference)
import functools

import jax
import jax.numpy as jnp
from jax import lax
from jax.experimental import pallas as pl
from jax.experimental.pallas import tpu as pltpu
from jax.experimental.pallas import tpu_sc as plsc

VOCAB = 205
D = 16
CHUNK = 1024
NB = 4
NC, NS = 2, 16
NW = NC * NS
UNROLL = 8


def _build(B):
    assert B % (NW * CHUNK) == 0
    per_w = B // NW
    nchunks = per_w // CHUNK
    assert nchunks % NB == 0
    nrounds = nchunks // NB
    assert nrounds >= 3

    mesh = plsc.VectorSubcoreMesh(core_axis_name="c", subcore_axis_name="s")

    @functools.partial(
        pl.kernel,
        out_type=jax.ShapeDtypeStruct((B * D,), jnp.float32),
        mesh=mesh,
        scratch_types=(
            [pltpu.VMEM((NB * CHUNK,), jnp.int32),
             pltpu.VMEM((NB * CHUNK * D,), jnp.float32),
             pltpu.VMEM((VOCAB * D,), jnp.float32)]
            + [pltpu.SemaphoreType.DMA] * NB
            + [pltpu.SemaphoreType.DMA] * NB
        ),
        compiler_params=pltpu.CompilerParams(
            use_tc_tiling_on_sc=False, needs_layout_passes=False),
    )
    def k(x_hbm, table_hbm, out_hbm, idx_v, rows_v, tbl_v, *sems):
        sem_i = sems[:NB]
        sem_w = sems[NB:]
        wid = lax.axis_index("s") * NC + lax.axis_index("c")
        row0 = wid * per_w

        pltpu.sync_copy(table_hbm, tbl_v)

        lane = lax.iota(jnp.int32, 16)

        def idx_load(g, b):
            return pltpu.make_async_copy(
                x_hbm.at[pl.ds(row0 + g * CHUNK, CHUNK)],
                idx_v.at[pl.ds(b * CHUNK, CHUNK)], sem_i[b])

        def write(g, b):
            return pltpu.make_async_copy(
                rows_v.at[pl.ds(b * CHUNK * D, CHUNK * D)],
                out_hbm.at[pl.ds((row0 + g * CHUNK) * D, CHUNK * D)],
                sem_w[b])

        def compute(b):
            def body(j, _):
                base = b * CHUNK + j * 16
                iv = idx_v[pl.ds(base, 16)] * D
                for r in range(16):
                    row = plsc.load_gather(tbl_v, [iv[r] + lane])
                    rows_v[pl.ds((base + r) * D, D)] = row
                return 0
            lax.fori_loop(0, CHUNK // 16, body, 0, unroll=UNROLL)

        for b in range(NB):
            idx_load(b, b).start()

        for b in range(NB):
            idx_load(b, b).wait()
            compute(b)
            write(b, b).start()
            idx_load(b + NB, b).start()

        def round_body(r, _):
            for b in range(NB):
                g = r * NB + b
                idx_load(g, b).wait()
                write(g - NB, b).wait()
                compute(b)
                write(g, b).start()
                idx_load(g + NB, b).start()
            return 0

        lax.fori_loop(1, nrounds - 1, round_body, 0)

        r = nrounds - 1
        for b in range(NB):
            g = r * NB + b
            idx_load(g, b).wait()
            write(g - NB, b).wait()
            compute(b)
            write(g, b).start()
        for b in range(NB):
            write(r * NB + b, b).wait()

    return k


def kernel(x, embed_site):
    B = x.size
    out = _build(B)(x.reshape(B).astype(jnp.int32), embed_site.reshape(-1))
    return out.reshape(B, D)
